# baseline (speedup 1.0000x reference)
import functools

import jax
import jax.numpy as jnp
from jax import lax
from jax.experimental import pallas as pl
from jax.experimental.pallas import tpu as pltpu
from jax.experimental.pallas import tpu_sc as plsc


def _sc_gather(idx_flat, table):
    seq = idx_flat.shape[0]
    d = table.shape[1]
    info = plsc.get_sparse_core_info()
    nw = info.num_cores * info.num_subcores
    b_per_w = seq // nw
    mesh = plsc.VectorSubcoreMesh(core_axis_name="c", subcore_axis_name="s")

    @functools.partial(
        pl.kernel,
        mesh=mesh,
        out_type=jax.ShapeDtypeStruct((seq, d), jnp.float32),
        scratch_types=[
            pltpu.VMEM((b_per_w,), jnp.int32),
            pltpu.VMEM((b_per_w, d), jnp.float32),
            pltpu.SemaphoreType.DMA,
        ],
    )
    def gather_kernel(idx_hbm, table_hbm, out_hbm, idx_v, rows_v, sem):
        wid = lax.axis_index("s") * info.num_cores + lax.axis_index("c")
        base = wid * b_per_w
        pltpu.sync_copy(idx_hbm.at[pl.ds(base, b_per_w)], idx_v)
        pltpu.async_copy(table_hbm.at[idx_v], rows_v, sem).wait()
        pltpu.sync_copy(rows_v, out_hbm.at[pl.ds(base, b_per_w)])

    return gather_kernel(idx_flat, table)


def _tc_matmul(x, w, tile_n=2048):
    m, k = x.shape
    n = w.shape[0]
    grid = pl.cdiv(n, tile_n)

    def body(x_ref, w_ref, o_ref):
        o_ref[...] = lax.dot_general(
            w_ref[...].astype(jnp.bfloat16), x_ref[...],
            (((1,), (1,)), ((), ())),
            preferred_element_type=jnp.float32,
        )

    return pl.pallas_call(
        body,
        grid=(grid,),
        in_specs=[
            pl.BlockSpec((m, k), lambda i: (0, 0)),
            pl.BlockSpec((tile_n, k), lambda i: (i, 0)),
        ],
        out_specs=pl.BlockSpec((tile_n, m), lambda i: (i, 0)),
        out_shape=jax.ShapeDtypeStruct((n, m), jnp.float32),
        compiler_params=pltpu.CompilerParams(
            vmem_limit_bytes=100 * 1024 * 1024,
        ),
    )(x, w)


def kernel(idx, wte, lm_head_w):
    b, s = idx.shape
    idx_flat = idx.reshape(-1).astype(jnp.int32)
    tok_emb = _sc_gather(idx_flat, wte)
    logits_t = _tc_matmul(tok_emb.astype(jnp.bfloat16), lm_head_w)
    return logits_t.T.reshape(b, s, lm_head_w.shape[0])


# speedup vs baseline: 1.0100x; 1.0100x over previous
import functools

import jax
import jax.numpy as jnp
from jax import lax
from jax.experimental import pallas as pl
from jax.experimental.pallas import tpu as pltpu
from jax.experimental.pallas import tpu_sc as plsc


def _sc_gather(idx_flat, table):
    seq = idx_flat.shape[0]
    d = table.shape[1]
    info = plsc.get_sparse_core_info()
    nw = info.num_cores * info.num_subcores
    b_per_w = seq // nw
    mesh = plsc.VectorSubcoreMesh(core_axis_name="c", subcore_axis_name="s")

    @functools.partial(
        pl.kernel,
        mesh=mesh,
        out_type=jax.ShapeDtypeStruct((seq, d), jnp.float32),
        scratch_types=[
            pltpu.VMEM((b_per_w,), jnp.int32),
            pltpu.VMEM((b_per_w, d), jnp.float32),
            pltpu.SemaphoreType.DMA,
        ],
    )
    def gather_kernel(idx_hbm, table_hbm, out_hbm, idx_v, rows_v, sem):
        wid = lax.axis_index("s") * info.num_cores + lax.axis_index("c")
        base = wid * b_per_w
        pltpu.sync_copy(idx_hbm.at[pl.ds(base, b_per_w)], idx_v)
        pltpu.async_copy(table_hbm.at[idx_v], rows_v, sem).wait()
        pltpu.sync_copy(rows_v, out_hbm.at[pl.ds(base, b_per_w)])

    return gather_kernel(idx_flat, table)


def _tc_matmul(x, w, tile_n=1856):
    m, k = x.shape
    n = w.shape[0]
    grid = pl.cdiv(n, tile_n)

    def body(x_ref, w_ref, o_ref, xb_ref):
        @pl.when(pl.program_id(0) == 0)
        def _():
            xb_ref[...] = x_ref[...].astype(jnp.bfloat16)

        o_ref[...] = lax.dot_general(
            w_ref[...].astype(jnp.bfloat16), xb_ref[...],
            (((1,), (1,)), ((), ())),
            preferred_element_type=jnp.float32,
        )

    return pl.pallas_call(
        body,
        grid=(grid,),
        in_specs=[
            pl.BlockSpec((m, k), lambda i: (0, 0)),
            pl.BlockSpec((tile_n, k), lambda i: (i, 0)),
        ],
        out_specs=pl.BlockSpec((tile_n, m), lambda i: (i, 0)),
        out_shape=jax.ShapeDtypeStruct((n, m), jnp.float32),
        scratch_shapes=[pltpu.VMEM((m, k), jnp.bfloat16)],
        compiler_params=pltpu.CompilerParams(
            vmem_limit_bytes=100 * 1024 * 1024,
        ),
    )(x, w)


def kernel(idx, wte, lm_head_w):
    b, s = idx.shape
    idx_flat = idx.reshape(-1).astype(jnp.int32)
    tok_emb = _sc_gather(idx_flat, wte)
    logits_t = _tc_matmul(tok_emb, lm_head_w)
    return logits_t.T.reshape(b, s, lm_head_w.shape[0])
